# baseline (speedup 1.0000x reference)
import functools

import jax
import jax.numpy as jnp
from jax import lax
from jax.experimental import pallas as pl
from jax.experimental.pallas import tpu as pltpu
from jax.experimental.pallas import tpu_sc as plsc

_N = 10000
_E = 320000
_H = 128
_C = 10
_L = 4

_NW = 32
_K = 128
_CPT = 80
_EPT = _CPT * _K
_EPAD = _NW * _EPT
_NPAD = 10112
_ZPT = _NPAD // 16


_G = 16
_CPT0 = 160
_NGRP = _CPT0 // _G


def _segsum_body(h_hbm, src_hbm, dst_hbm, out_hbm,
                 srcv, dstv, b0, b1, zb, acc, g0, g1, s0, s1):
    c = lax.axis_index("c")
    s = lax.axis_index("s")

    @pl.when(c == 0)
    def _():
        base = s * _CPT0
        z16 = jnp.zeros((16,), jnp.float32)
        for r in range(64):
            for q in range(_H // 16):
                zb[r, pl.ds(q * 16, 16)] = z16
        for i in range(9):
            pltpu.async_copy(zb, acc.at[pl.ds(s * _ZPT + i * 64, 64)], g0)
        pltpu.async_copy(zb.at[pl.ds(0, 56)],
                         acc.at[pl.ds(s * _ZPT + 576, 56)], g0)
        for i in range(9):
            pltpu.make_async_copy(zb, acc.at[pl.ds(s * _ZPT, 64)], g0).wait()
        pltpu.make_async_copy(zb.at[pl.ds(0, 56)],
                              acc.at[pl.ds(s * _ZPT, 56)], g0).wait()
        plsc.subcore_barrier()

        bufs = (b0, b1)
        gsems = (g0, g1)
        ssems = (s0, s1)

        def wait_gather(b):
            pltpu.make_async_copy(h_hbm.at[srcv.at[0]], bufs[b],
                                  gsems[b]).wait()

        def wait_scatter(b):
            pltpu.make_async_copy(bufs[b], acc.at[dstv.at[0]],
                                  ssems[b]).wait()

        def group(grp, carry):
            gb = base + grp * _G
            pltpu.sync_copy(src_hbm.at[pl.ds(gb, _G)], srcv)
            pltpu.sync_copy(dst_hbm.at[pl.ds(gb, _G)], dstv)
            for b in range(2):
                pltpu.async_copy(h_hbm.at[srcv.at[b]], bufs[b], gsems[b])

            def body(j, carry2):
                cb = j * 2
                for b in range(2):
                    wait_gather(b)
                    pltpu.async_copy(bufs[b], acc.at[dstv.at[cb + b]],
                                     ssems[b], add=True)

                @pl.when(j < _G // 2 - 1)
                def _():
                    for b in range(2):
                        wait_scatter(b)
                        pltpu.async_copy(h_hbm.at[srcv.at[cb + 2 + b]],
                                         bufs[b], gsems[b])
                return carry2

            lax.fori_loop(0, _G // 2, body, 0)
            for b in range(2):
                wait_scatter(b)
            return carry

        lax.fori_loop(0, _NGRP, group, 0)

        plsc.subcore_barrier()
        pltpu.sync_copy(acc.at[pl.ds(s * _ZPT, _ZPT)],
                        out_hbm.at[pl.ds(s * _ZPT, _ZPT)])


_segsum = functools.partial(
    pl.kernel,
    out_type=jax.ShapeDtypeStruct((_NPAD, _H), jnp.float32),
    mesh=plsc.VectorSubcoreMesh(core_axis_name="c", subcore_axis_name="s",
                                num_cores=2, num_subcores=16),
    scratch_types=[
        pltpu.VMEM((_G, _K), jnp.int32),
        pltpu.VMEM((_G, _K), jnp.int32),
        pltpu.VMEM((_K, _H), jnp.float32),
        pltpu.VMEM((_K, _H), jnp.float32),
        pltpu.VMEM((64, _H), jnp.float32),
        pltpu.VMEM_SHARED((_NPAD, _H), jnp.float32),
        pltpu.SemaphoreType.DMA,
        pltpu.SemaphoreType.DMA,
        pltpu.SemaphoreType.DMA,
        pltpu.SemaphoreType.DMA,
    ],
)(_segsum_body)


def _embed_body(h_ref, w_ref, m_ref, b_ref, out_ref, cs_ref):
    wm = w_ref[...] * m_ref[...]
    out = jnp.dot(h_ref[...], wm.T, preferred_element_type=jnp.float32)
    out = out + b_ref[...]
    out_ref[...] = out
    cs_ref[...] = jnp.sum(out, axis=0, keepdims=True)


def _layer_body(eps_ref, h_ref, p_ref, w1_ref, m1_ref, b1_ref,
                g1_ref, be1_ref, w2_ref, m2_ref, b2_ref, gn_ref, bnb_ref,
                sn_ref, out_ref, cs_ref):
    h = h_ref[...]
    neigh = p_ref[:_N, :]
    z = (1.0 + eps_ref[0]) * h + neigh
    w1m = w1_ref[...] * m1_ref[...]
    z = jnp.dot(z, w1m.T, preferred_element_type=jnp.float32) + b1_ref[...]
    mu = jnp.mean(z, axis=0, keepdims=True)
    var = jnp.mean((z - mu) * (z - mu), axis=0, keepdims=True)
    z = g1_ref[...] * (z - mu) * lax.rsqrt(var + 1e-5) + be1_ref[...]
    z = jnp.maximum(z, 0.0)
    w2m = w2_ref[...] * m2_ref[...]
    z = jnp.dot(z, w2m.T, preferred_element_type=jnp.float32) + b2_ref[...]
    z = z * sn_ref[...]
    mu2 = jnp.mean(z, axis=0, keepdims=True)
    var2 = jnp.mean((z - mu2) * (z - mu2), axis=0, keepdims=True)
    z = gn_ref[...] * (z - mu2) * lax.rsqrt(var2 + 1e-5) + bnb_ref[...]
    z = jnp.maximum(z, 0.0)
    out = h + z
    out_ref[...] = out
    cs_ref[...] = jnp.sum(out, axis=0, keepdims=True)


def _readout_body(cs_ref, wp_ref, bp_ref, out_ref):
    acc = jnp.zeros((1, _C), dtype=jnp.float32)
    for i in range(_L + 1):
        pooled = cs_ref[i:i + 1, :] * (1.0 / _N)
        acc = acc + jnp.dot(pooled, wp_ref[i].T,
                            preferred_element_type=jnp.float32)
        acc = acc + bp_ref[i:i + 1, :]
    out_ref[...] = acc


def kernel(h, edge_index, e, snorm_n, snorm_e, W_emb, M_emb, b_emb, eps,
           W1, M1, b1, g1, be1, W2, M2, b2, gn, bnb, Wp, bp):
    f32 = jnp.float32
    src = edge_index[0]
    dst = edge_index[1]
    pad = _EPAD - _E
    src_p = jnp.concatenate([src, jnp.zeros((pad,), jnp.int32)])
    src_p = src_p.reshape(_NW * _CPT, _K)
    pad_dst = _N + jnp.arange(pad, dtype=jnp.int32) % (_NPAD - _N)
    dst_p = jnp.concatenate([dst, pad_dst])
    dst_p = dst_p.reshape(_NW * _CPT, _K)

    vmem = pl.BlockSpec(memory_space=pltpu.MemorySpace.VMEM)
    smem = pl.BlockSpec(memory_space=pltpu.MemorySpace.SMEM)

    embed = pl.pallas_call(
        _embed_body,
        out_shape=(jax.ShapeDtypeStruct((_N, _H), f32),
                   jax.ShapeDtypeStruct((1, _H), f32)),
        in_specs=[vmem] * 4,
        out_specs=(vmem, vmem),
    )
    hh, cs0 = embed(h, W_emb, M_emb, b_emb.reshape(1, _H))

    layer = pl.pallas_call(
        _layer_body,
        out_shape=(jax.ShapeDtypeStruct((_N, _H), f32),
                   jax.ShapeDtypeStruct((1, _H), f32)),
        in_specs=[smem] + [vmem] * 13,
        out_specs=(vmem, vmem),
    )

    colsums = [cs0]
    for i in range(_L):
        parts = _segsum(hh, src_p, dst_p)
        hh, csi = layer(eps[i:i + 1], hh, parts,
                        W1[i], M1[i], b1[i:i + 1], g1[i:i + 1],
                        be1[i:i + 1], W2[i], M2[i], b2[i:i + 1],
                        gn[i:i + 1], bnb[i:i + 1], snorm_n)
        colsums.append(csi)

    readout = pl.pallas_call(
        _readout_body,
        out_shape=jax.ShapeDtypeStruct((1, _C), f32),
        in_specs=[vmem] * 3,
        out_specs=vmem,
    )
    return readout(jnp.concatenate(colsums, axis=0), Wp, bp)


# speedup vs baseline: 1.6426x; 1.6426x over previous
import functools

import jax
import jax.numpy as jnp
from jax import lax
from jax.experimental import pallas as pl
from jax.experimental.pallas import tpu as pltpu
from jax.experimental.pallas import tpu_sc as plsc

_N = 10000
_E = 320000
_H = 128
_C = 10
_L = 4

_NW = 32
_K = 128
_CPT = 80
_EPT = _CPT * _K
_EPAD = _NW * _EPT
_NPAD = 10112
_ZPT = _NPAD // 16


_G = 8
_CPT0 = 152
_CPT1 = 8


def _segsum_body(h_hbm, src_hbm, dst_hbm, out_hbm,
                 srcv, dstv, b0, b1, zb, acc, g0, g1, s0, s1):
    c = lax.axis_index("c")
    s = lax.axis_index("s")
    base = lax.select(c == 0, s * _CPT0, 16 * _CPT0 + s * _CPT1)
    ngrp = lax.select(c == 0, _CPT0 // _G, _CPT1 // _G)
    z16 = jnp.zeros((16,), jnp.float32)
    for r in range(64):
        for q in range(_H // 16):
            zb[r, pl.ds(q * 16, 16)] = z16
    for i in range(9):
        pltpu.async_copy(zb, acc.at[pl.ds(s * _ZPT + i * 64, 64)], g0)
    pltpu.async_copy(zb.at[pl.ds(0, 56)],
                     acc.at[pl.ds(s * _ZPT + 576, 56)], g0)
    for i in range(9):
        pltpu.make_async_copy(zb, acc.at[pl.ds(s * _ZPT, 64)], g0).wait()
    pltpu.make_async_copy(zb.at[pl.ds(0, 56)],
                          acc.at[pl.ds(s * _ZPT, 56)], g0).wait()
    plsc.subcore_barrier()

    bufs = (b0, b1)
    gsems = (g0, g1)
    ssems = (s0, s1)

    def wait_gather(b):
        pltpu.make_async_copy(h_hbm.at[srcv.at[0]], bufs[b], gsems[b]).wait()

    def wait_scatter(b):
        pltpu.make_async_copy(bufs[b], acc.at[dstv.at[0]], ssems[b]).wait()

    def group(grp, carry):
        gb = base + grp * _G
        pltpu.sync_copy(src_hbm.at[pl.ds(gb, _G)], srcv)
        pltpu.sync_copy(dst_hbm.at[pl.ds(gb, _G)], dstv)
        for b in range(2):
            pltpu.async_copy(h_hbm.at[srcv.at[b]], bufs[b], gsems[b])

        def body(j, carry2):
            cb = j * 2
            for b in range(2):
                wait_gather(b)
                pltpu.async_copy(bufs[b], acc.at[dstv.at[cb + b]],
                                 ssems[b], add=True)

            @pl.when(j < _G // 2 - 1)
            def _():
                for b in range(2):
                    wait_scatter(b)
                    pltpu.async_copy(h_hbm.at[srcv.at[cb + 2 + b]],
                                     bufs[b], gsems[b])
            return carry2

        lax.fori_loop(0, _G // 2, body, 0)
        for b in range(2):
            wait_scatter(b)
        return carry

    lax.fori_loop(0, ngrp, group, 0)

    plsc.subcore_barrier()
    pltpu.sync_copy(acc.at[pl.ds(s * _ZPT, _ZPT)],
                    out_hbm.at[c, pl.ds(s * _ZPT, _ZPT)])


_segsum = functools.partial(
    pl.kernel,
    out_type=jax.ShapeDtypeStruct((2, _NPAD, _H), jnp.float32),
    mesh=plsc.VectorSubcoreMesh(core_axis_name="c", subcore_axis_name="s",
                                num_cores=2, num_subcores=16),
    scratch_types=[
        pltpu.VMEM((_G, _K), jnp.int32),
        pltpu.VMEM((_G, _K), jnp.int32),
        pltpu.VMEM((_K, _H), jnp.float32),
        pltpu.VMEM((_K, _H), jnp.float32),
        pltpu.VMEM((64, _H), jnp.float32),
        pltpu.VMEM_SHARED((_NPAD, _H), jnp.float32),
        pltpu.SemaphoreType.DMA,
        pltpu.SemaphoreType.DMA,
        pltpu.SemaphoreType.DMA,
        pltpu.SemaphoreType.DMA,
    ],
)(_segsum_body)


def _embed_body(h_ref, w_ref, m_ref, b_ref, out_ref, cs_ref):
    wm = w_ref[...] * m_ref[...]
    out = jnp.dot(h_ref[...], wm.T, preferred_element_type=jnp.float32)
    out = out + b_ref[...]
    out_ref[...] = out
    cs_ref[...] = jnp.sum(out, axis=0, keepdims=True)


def _layer_body(eps_ref, h_ref, p_ref, w1_ref, m1_ref, b1_ref,
                g1_ref, be1_ref, w2_ref, m2_ref, b2_ref, gn_ref, bnb_ref,
                sn_ref, out_ref, cs_ref):
    h = h_ref[...]
    neigh = p_ref[0, :_N, :] + p_ref[1, :_N, :]
    z = (1.0 + eps_ref[0]) * h + neigh
    w1m = w1_ref[...] * m1_ref[...]
    z = jnp.dot(z, w1m.T, preferred_element_type=jnp.float32) + b1_ref[...]
    mu = jnp.mean(z, axis=0, keepdims=True)
    var = jnp.mean((z - mu) * (z - mu), axis=0, keepdims=True)
    z = g1_ref[...] * (z - mu) * lax.rsqrt(var + 1e-5) + be1_ref[...]
    z = jnp.maximum(z, 0.0)
    w2m = w2_ref[...] * m2_ref[...]
    z = jnp.dot(z, w2m.T, preferred_element_type=jnp.float32) + b2_ref[...]
    z = z * sn_ref[...]
    mu2 = jnp.mean(z, axis=0, keepdims=True)
    var2 = jnp.mean((z - mu2) * (z - mu2), axis=0, keepdims=True)
    z = gn_ref[...] * (z - mu2) * lax.rsqrt(var2 + 1e-5) + bnb_ref[...]
    z = jnp.maximum(z, 0.0)
    out = h + z
    out_ref[...] = out
    cs_ref[...] = jnp.sum(out, axis=0, keepdims=True)


def _readout_body(cs_ref, wp_ref, bp_ref, out_ref):
    acc = jnp.zeros((1, _C), dtype=jnp.float32)
    for i in range(_L + 1):
        pooled = cs_ref[i:i + 1, :] * (1.0 / _N)
        acc = acc + jnp.dot(pooled, wp_ref[i].T,
                            preferred_element_type=jnp.float32)
        acc = acc + bp_ref[i:i + 1, :]
    out_ref[...] = acc


def kernel(h, edge_index, e, snorm_n, snorm_e, W_emb, M_emb, b_emb, eps,
           W1, M1, b1, g1, be1, W2, M2, b2, gn, bnb, Wp, bp):
    f32 = jnp.float32
    src = edge_index[0]
    dst = edge_index[1]
    pad = _EPAD - _E
    src_p = jnp.concatenate([src, jnp.zeros((pad,), jnp.int32)])
    src_p = src_p.reshape(_NW * _CPT, _K)
    pad_dst = _N + jnp.arange(pad, dtype=jnp.int32) % (_NPAD - _N)
    dst_p = jnp.concatenate([dst, pad_dst])
    dst_p = dst_p.reshape(_NW * _CPT, _K)

    vmem = pl.BlockSpec(memory_space=pltpu.MemorySpace.VMEM)
    smem = pl.BlockSpec(memory_space=pltpu.MemorySpace.SMEM)

    embed = pl.pallas_call(
        _embed_body,
        out_shape=(jax.ShapeDtypeStruct((_N, _H), f32),
                   jax.ShapeDtypeStruct((1, _H), f32)),
        in_specs=[vmem] * 4,
        out_specs=(vmem, vmem),
    )
    hh, cs0 = embed(h, W_emb, M_emb, b_emb.reshape(1, _H))

    layer = pl.pallas_call(
        _layer_body,
        out_shape=(jax.ShapeDtypeStruct((_N, _H), f32),
                   jax.ShapeDtypeStruct((1, _H), f32)),
        in_specs=[smem] + [vmem] * 13,
        out_specs=(vmem, vmem),
    )

    colsums = [cs0]
    for i in range(_L):
        parts = _segsum(hh, src_p, dst_p)
        hh, csi = layer(eps[i:i + 1], hh, parts,
                        W1[i], M1[i], b1[i:i + 1], g1[i:i + 1],
                        be1[i:i + 1], W2[i], M2[i], b2[i:i + 1],
                        gn[i:i + 1], bnb[i:i + 1], snorm_n)
        colsums.append(csi)

    readout = pl.pallas_call(
        _readout_body,
        out_shape=jax.ShapeDtypeStruct((1, _C), f32),
        in_specs=[vmem] * 3,
        out_specs=vmem,
    )
    return readout(jnp.concatenate(colsums, axis=0), Wp, bp)
